# baseline (device time: 58632 ns/iter reference)
import jax
import jax.numpy as jnp
from jax import lax
from jax.experimental import pallas as pl
from jax.experimental.pallas import tpu as pltpu

B, H, D, BS = 8, 8, 128, 16
NPAGES_LOCAL = 512
NPB = 32
NBLK = NPAGES_LOCAL // NPB
T = NPB * BS
NEG = -1e30
SCALE = D ** -0.5


def _body(q_ref, k_ref, v_ref, btv_ref, out_ref,
          acc_ref, m_ref, l_ref,
          acc_comm, ml_send, ml_comm, send_sems, recv_sems):
    i = pl.program_id(0)
    my_x = lax.axis_index("x")
    my_y = lax.axis_index("y")
    nbr = (my_x, 1 - my_y)

    @pl.when(i == 0)
    def _init():
        barrier = pltpu.get_barrier_semaphore()
        pl.semaphore_signal(barrier, inc=1, device_id=nbr,
                            device_id_type=pl.DeviceIdType.MESH)
        pl.semaphore_wait(barrier, 1)
        m_ref[...] = jnp.full((B, H), NEG, jnp.float32)
        l_ref[...] = jnp.zeros((B, H), jnp.float32)
        acc_ref[...] = jnp.zeros((B, H, D), jnp.float32)

    base = my_y * NPAGES_LOCAL + i * NPB
    pidc = base + lax.broadcasted_iota(jnp.int32, (1, NPB, 1), 1)
    btv = btv_ref[...]
    counts = jnp.sum((btv[:, None, :] == pidc).astype(jnp.float32),
                     axis=2)
    w = jnp.broadcast_to(counts[:, :, None], (B, NPB, BS)).reshape(B, T)

    q = q_ref[...].reshape(B, H, D)
    kr = k_ref[...].reshape(T, H, D)
    vr = v_ref[...].reshape(T, H, D)

    s_list = []
    for h in range(H):
        s_list.append(lax.dot_general(
            q[:, h, :], kr[:, h, :],
            (((1,), (1,)), ((), ())),
            preferred_element_type=jnp.float32))
    s = jnp.stack(s_list, axis=1) * SCALE
    s = jnp.where((w > 0.0)[:, None, :], s, NEG)

    m_prev = m_ref[...]
    m_new = jnp.maximum(m_prev, jnp.max(s, axis=2))
    alpha = jnp.exp(m_prev - m_new)
    p = jnp.exp(s - m_new[:, :, None]) * w[:, None, :]
    m_ref[...] = m_new
    l_ref[...] = alpha * l_ref[...] + jnp.sum(p, axis=2)

    o_list = []
    for h in range(H):
        o_list.append(lax.dot_general(
            p[:, h, :], vr[:, h, :],
            (((1,), (0,)), ((), ())),
            preferred_element_type=jnp.float32))
    o = jnp.stack(o_list, axis=1)
    acc_ref[...] = alpha[:, :, None] * acc_ref[...] + o

    @pl.when(i == NBLK - 1)
    def _finish():
        ml_send[0, :, :] = m_ref[...]
        ml_send[1, :, :] = l_ref[...]
        rdma_acc = pltpu.make_async_remote_copy(
            src_ref=acc_ref, dst_ref=acc_comm,
            send_sem=send_sems.at[0], recv_sem=recv_sems.at[0],
            device_id=nbr, device_id_type=pl.DeviceIdType.MESH)
        rdma_ml = pltpu.make_async_remote_copy(
            src_ref=ml_send, dst_ref=ml_comm,
            send_sem=send_sems.at[1], recv_sem=recv_sems.at[1],
            device_id=nbr, device_id_type=pl.DeviceIdType.MESH)
        rdma_acc.start()
        rdma_ml.start()
        rdma_acc.wait()
        rdma_ml.wait()

        m_loc = m_ref[...]
        m_rem = ml_comm[0, :, :]
        l_rem = ml_comm[1, :, :]
        m_f = jnp.maximum(m_loc, m_rem)
        a_loc = jnp.exp(m_loc - m_f)
        a_rem = jnp.exp(m_rem - m_f)
        l_f = a_loc * l_ref[...] + a_rem * l_rem
        acc_f = (a_loc[:, :, None] * acc_ref[...]
                 + a_rem[:, :, None] * acc_comm[...])
        out_ref[...] = (acc_f / l_f[:, :, None]).reshape(B, 1, H, D)


def kernel(Q, K, V, bt, lens):
    jidx = lax.broadcasted_iota(jnp.int32, (B, NPAGES_LOCAL), 1)
    btv = jnp.where(jidx < lens[:, None], bt, -1)

    return pl.pallas_call(
        _body,
        grid=(NBLK,),
        in_specs=[
            pl.BlockSpec((B, 1, H, D), lambda i: (0, 0, 0, 0)),
            pl.BlockSpec((NPB, BS, H, D), lambda i: (i, 0, 0, 0)),
            pl.BlockSpec((NPB, BS, H, D), lambda i: (i, 0, 0, 0)),
            pl.BlockSpec((B, NPAGES_LOCAL), lambda i: (0, 0)),
        ],
        out_specs=pl.BlockSpec((B, 1, H, D), lambda i: (0, 0, 0, 0)),
        out_shape=jax.ShapeDtypeStruct((B, 1, H, D), jnp.float32),
        scratch_shapes=[
            pltpu.VMEM((B, H, D), jnp.float32),
            pltpu.VMEM((B, H), jnp.float32),
            pltpu.VMEM((B, H), jnp.float32),
            pltpu.VMEM((B, H, D), jnp.float32),
            pltpu.VMEM((2, B, H), jnp.float32),
            pltpu.VMEM((2, B, H), jnp.float32),
            pltpu.SemaphoreType.DMA((2,)),
            pltpu.SemaphoreType.DMA((2,)),
        ],
        compiler_params=pltpu.CompilerParams(
            collective_id=0,
            dimension_semantics=("arbitrary",),
        ),
    )(Q, K, V, btv)


# device time: 31132 ns/iter; 1.8833x vs baseline; 1.8833x over previous
import jax
import jax.numpy as jnp
from jax import lax
from jax.experimental import pallas as pl
from jax.experimental.pallas import tpu as pltpu

B, H, D, BS = 8, 8, 128, 16
NPAGES_LOCAL = 512
NPB = 32
NBLK = NPAGES_LOCAL // NPB // 2
T = NPB * BS
NEG = -1e30
SCALE = D ** -0.5


def _merge_exchange(stage, nbr, acc_ref, m_ref, l_ref,
                    acc_comm, ml_send, ml_comm, send_sems, recv_sems):
    s_a, s_m = 2 * stage, 2 * stage + 1
    ml_send[0, :, :] = m_ref[...]
    ml_send[1, :, :] = l_ref[...]
    rdma_acc = pltpu.make_async_remote_copy(
        src_ref=acc_ref, dst_ref=acc_comm,
        send_sem=send_sems.at[s_a], recv_sem=recv_sems.at[s_a],
        device_id=nbr, device_id_type=pl.DeviceIdType.MESH)
    rdma_ml = pltpu.make_async_remote_copy(
        src_ref=ml_send, dst_ref=ml_comm,
        send_sem=send_sems.at[s_m], recv_sem=recv_sems.at[s_m],
        device_id=nbr, device_id_type=pl.DeviceIdType.MESH)
    rdma_acc.start()
    rdma_ml.start()
    rdma_acc.wait()
    rdma_ml.wait()

    m_loc = m_ref[...]
    m_rem = ml_comm[0, :, :]
    l_rem = ml_comm[1, :, :]
    m_f = jnp.maximum(m_loc, m_rem)
    a_loc = jnp.exp(m_loc - m_f)
    a_rem = jnp.exp(m_rem - m_f)
    m_ref[...] = m_f
    l_ref[...] = a_loc * l_ref[...] + a_rem * l_rem
    acc_ref[...] = (a_loc[:, :, None] * acc_ref[...]
                    + a_rem[:, :, None] * acc_comm[...])


def _body(x_off_ref, q_ref, k_ref, v_ref, btv_ref, out_ref,
          acc_ref, m_ref, l_ref,
          acc_comm, ml_send, ml_comm, send_sems, recv_sems):
    i = pl.program_id(0)
    my_x = lax.axis_index("x")
    my_y = lax.axis_index("y")
    nbr_y = (my_x, 1 - my_y)
    nbr_x = (1 - my_x, my_y)

    @pl.when(i == 0)
    def _init():
        barrier = pltpu.get_barrier_semaphore()
        for nbr in (nbr_y, nbr_x):
            pl.semaphore_signal(barrier, inc=1, device_id=nbr,
                                device_id_type=pl.DeviceIdType.MESH)
        pl.semaphore_wait(barrier, 2)
        m_ref[...] = jnp.full((H, B), NEG, jnp.float32)
        l_ref[...] = jnp.zeros((H, B), jnp.float32)
        acc_ref[...] = jnp.zeros((H, B, D), jnp.float32)

    base = my_y * NPAGES_LOCAL + (my_x * NBLK + i) * NPB
    pidc = base + lax.broadcasted_iota(jnp.int32, (1, NPB, 1), 1)
    btv = btv_ref[...]
    counts = jnp.sum((btv[:, None, :] == pidc).astype(jnp.float32),
                     axis=2)
    w = jnp.broadcast_to(counts[:, :, None], (B, NPB, BS)).reshape(B, T)

    q = q_ref[...].reshape(B, H, D)
    kr = k_ref[...].reshape(T, H, D)
    vr = v_ref[...].reshape(T, H, D)

    s_list = []
    for h in range(H):
        s_list.append(lax.dot_general(
            q[:, h, :], kr[:, h, :],
            (((1,), (1,)), ((), ())),
            preferred_element_type=jnp.float32))
    s = jnp.stack(s_list, axis=0) * SCALE
    s = jnp.where((w > 0.0)[None, :, :], s, NEG)

    m_prev = m_ref[...]
    m_new = jnp.maximum(m_prev, jnp.max(s, axis=2))
    alpha = jnp.exp(m_prev - m_new)
    p = jnp.exp(s - m_new[:, :, None]) * w[None, :, :]
    m_ref[...] = m_new
    l_ref[...] = alpha * l_ref[...] + jnp.sum(p, axis=2)

    o_list = []
    for h in range(H):
        o_list.append(lax.dot_general(
            p[h], vr[:, h, :],
            (((1,), (0,)), ((), ())),
            preferred_element_type=jnp.float32))
    o = jnp.stack(o_list, axis=0)
    acc_ref[...] = alpha[:, :, None] * acc_ref[...] + o

    @pl.when(i == NBLK - 1)
    def _finish():
        _merge_exchange(0, nbr_y, acc_ref, m_ref, l_ref,
                        acc_comm, ml_send, ml_comm, send_sems, recv_sems)
        _merge_exchange(1, nbr_x, acc_ref, m_ref, l_ref,
                        acc_comm, ml_send, ml_comm, send_sems, recv_sems)

        out = acc_ref[...] / l_ref[...][:, :, None]
        out_ref[...] = jnp.transpose(out, (1, 0, 2)).reshape(B, 1, H, D)


def kernel(Q, K, V, bt, lens):
    jidx = lax.broadcasted_iota(jnp.int32, (B, NPAGES_LOCAL), 1)
    btv = jnp.where(jidx < lens[:, None], bt, -1)

    x_off = (lax.axis_index("x") * NBLK).astype(jnp.int32).reshape(1)

    grid_spec = pltpu.PrefetchScalarGridSpec(
        num_scalar_prefetch=1,
        grid=(NBLK,),
        in_specs=[
            pl.BlockSpec((B, 1, H, D), lambda i, xo: (0, 0, 0, 0)),
            pl.BlockSpec((NPB, BS, H, D), lambda i, xo: (xo[0] + i, 0, 0, 0)),
            pl.BlockSpec((NPB, BS, H, D), lambda i, xo: (xo[0] + i, 0, 0, 0)),
            pl.BlockSpec((B, NPAGES_LOCAL), lambda i, xo: (0, 0)),
        ],
        out_specs=pl.BlockSpec((B, 1, H, D), lambda i, xo: (0, 0, 0, 0)),
        scratch_shapes=[
            pltpu.VMEM((H, B, D), jnp.float32),
            pltpu.VMEM((H, B), jnp.float32),
            pltpu.VMEM((H, B), jnp.float32),
            pltpu.VMEM((H, B, D), jnp.float32),
            pltpu.VMEM((2, H, B), jnp.float32),
            pltpu.VMEM((2, H, B), jnp.float32),
            pltpu.SemaphoreType.DMA((4,)),
            pltpu.SemaphoreType.DMA((4,)),
        ],
    )

    return pl.pallas_call(
        _body,
        grid_spec=grid_spec,
        out_shape=jax.ShapeDtypeStruct((B, 1, H, D), jnp.float32),
        compiler_params=pltpu.CompilerParams(
            collective_id=0,
            dimension_semantics=("arbitrary",),
        ),
    )(x_off, Q, K, V, btv)
